# baseline (speedup 1.0000x reference)
import functools

import jax
import jax.numpy as jnp
from jax import lax
from jax.experimental import pallas as pl
from jax.experimental.pallas import tpu as pltpu
from jax.experimental.pallas import tpu_sc as plsc

NC = 2
NS = 16
NW = NC * NS
EMB = 32
HID = 64
OUT = 32
UNIT = 128
LANE = 128
GRP = LANE // EMB
MROW = 1024
BBT = GRP * MROW


def _make_format(table_rows: int):
    ntile = table_rows // 128
    tail = table_rows - ntile * 128
    mesh = plsc.VectorSubcoreMesh(
        core_axis_name="c", subcore_axis_name="s", num_cores=NC, num_subcores=NS
    )

    @functools.partial(
        pl.kernel,
        out_type=jax.ShapeDtypeStruct((table_rows * EMB // LANE, LANE), jnp.float32),
        mesh=mesh,
        scratch_types=[
            pltpu.VMEM((2, EMB, LANE), jnp.float32),
            pltpu.VMEM((2, EMB, LANE), jnp.float32),
            pltpu.SemaphoreType.DMA,
            pltpu.SemaphoreType.DMA,
        ],
        compiler_params=pltpu.CompilerParams(
            use_tc_tiling_on_sc=True, needs_layout_passes=False
        ),
    )
    def fmt_k(src_hbm, tail_hbm, out_hbm, tile_v, tr_v, sem0, sem1):
        wid = lax.axis_index("s") * NC + lax.axis_index("c")
        sems = (sem0, sem1)
        iota = lax.iota(jnp.int32, 16)
        nfull = ntile // NW
        nextra = ntile - nfull * NW

        def tile_of(i):
            return wid + i * NW

        def transpose_tile(b):
            for q in range(EMB):
                for p in range(8):
                    c = p // 2
                    f_idx = iota + (p % 2) * 16
                    l_idx = jnp.broadcast_to(4 * q + c, (16,)).astype(jnp.int32)
                    vals = plsc.load_gather(tile_v.at[b], [f_idx, l_idx])
                    tr_v[b, q, pl.ds(p * 16, 16)] = vals

        def issue(i, b):
            t = tile_of(i)
            pltpu.async_copy(
                src_hbm.at[:, pl.ds(t * LANE, LANE)], tile_v.at[b], sems[b]
            )

        def wait(i, b):
            t = tile_of(i)
            pltpu.make_async_copy(
                src_hbm.at[:, pl.ds(t * LANE, LANE)], tile_v.at[b], sems[b]
            ).wait()

        def flush(i, b):
            t = tile_of(i)
            pltpu.sync_copy(
                tr_v.at[b], out_hbm.at[pl.ds(t * EMB, EMB)]
            )

        has_extra = wid < nextra
        issue(0, 0)
        issue(1, 1)

        def body(tpair, carry):
            for b in range(2):
                i = 2 * tpair + b
                wait(i, b)
                transpose_tile(b)

                @pl.when((i + 2 < nfull) | ((i + 2 == nfull) & has_extra))
                def _():
                    issue(i + 2, b)

                flush(i, b)
            return carry

        lax.fori_loop(0, nfull // 2, body, 0)

        @pl.when(has_extra)
        def _():
            wait(nfull, 0)
            transpose_tile(0)
            flush(nfull, 0)

        if tail:
            @pl.when(wid == 0)
            def _():
                pltpu.sync_copy(
                    tail_hbm,
                    out_hbm.at[pl.ds(ntile * EMB, tail * EMB // LANE)],
                )

    return fmt_k


def _make_gather(batch: int, hist: int):
    n_rows = batch * hist
    cols_per_w = batch // NW
    tunits = cols_per_w // UNIT
    nunit = hist * tunits
    nj = batch // BBT
    mesh = plsc.VectorSubcoreMesh(
        core_axis_name="c", subcore_axis_name="s", num_cores=NC, num_subcores=NS
    )

    @functools.partial(
        pl.kernel,
        out_type=jax.ShapeDtypeStruct((n_rows // GRP, LANE), jnp.float32),
        mesh=mesh,
        scratch_types=[
            pltpu.VMEM((hist, cols_per_w), jnp.int32),
            pltpu.VMEM((2, UNIT, EMB), jnp.float32),
            pltpu.SemaphoreType.DMA,
            pltpu.SemaphoreType.DMA,
        ],
        compiler_params=pltpu.CompilerParams(use_tc_tiling_on_sc=False),
    )
    def gather_k(table_hbm, idx_hbm, out_hbm, idx_v, rows_v, sem0, sem1):
        wid = lax.axis_index("s") * NC + lax.axis_index("c")
        col0 = wid * cols_per_w
        jblk = wid // 8
        jgrp = (wid % 8) // 2
        half = (wid % 2) * 512
        lane0 = jgrp * EMB
        pltpu.sync_copy(idx_hbm.at[:, pl.ds(col0, cols_per_w)], idx_v)
        sems = (sem0, sem1)

        def unit_src(u):
            l = u // tunits
            t = u % tunits
            return table_hbm.at[idx_v.at[l, pl.ds(t * UNIT, UNIT)]]

        def unit_dst(u):
            l = u // tunits
            t = u % tunits
            q0 = (l * nj + jblk) * MROW + half + t * UNIT
            return out_hbm.at[pl.ds(q0, UNIT), pl.ds(lane0, EMB)]

        for b in range(2):
            pltpu.async_copy(unit_src(b), rows_v.at[b], sems[b])

        def body(t, carry):
            for b in range(2):
                u = 2 * t + b
                pltpu.make_async_copy(unit_src(u), rows_v.at[b], sems[b]).wait()
                pltpu.sync_copy(rows_v.at[b], unit_dst(u))
                pltpu.async_copy(unit_src(u + 2), rows_v.at[b], sems[b])
            return carry

        lax.fori_loop(0, nunit // 2 - 1, body, 0)

        for b in range(2):
            u = nunit - 2 + b
            pltpu.make_async_copy(unit_src(u), rows_v.at[b], sems[b]).wait()
            pltpu.sync_copy(rows_v.at[b], unit_dst(u))

    return gather_k


def _mlp_body(x_ref, w1_ref, b1_ref, w2_ref, b2_ref, o_ref):
    xp = x_ref[...]
    w1 = w1_ref[...]
    b1 = b1_ref[...]
    w2 = w2_ref[...]
    b2 = b2_ref[...]
    for j in range(GRP):
        x = xp[:, j * EMB:(j + 1) * EMB]
        pre1 = jnp.dot(x, w1, preferred_element_type=jnp.float32) + b1
        h = 1.0 / (1.0 + jnp.exp(-pre1))
        pre2 = jnp.dot(h, w2, preferred_element_type=jnp.float32) + b2
        y = 1.0 / (1.0 + jnp.exp(-pre2))
        o_ref[0, :, pl.ds(j * MROW, MROW)] = y.T


def _mlp(x_packed, W1, b1, W2, b2, batch, hist):
    nj = batch // BBT
    return pl.pallas_call(
        _mlp_body,
        grid=(hist, nj),
        in_specs=[
            pl.BlockSpec((MROW, LANE), lambda l, j: (l * nj + j, 0)),
            pl.BlockSpec((EMB, HID), lambda l, j: (0, 0)),
            pl.BlockSpec((1, HID), lambda l, j: (0, 0)),
            pl.BlockSpec((HID, OUT), lambda l, j: (0, 0)),
            pl.BlockSpec((1, OUT), lambda l, j: (0, 0)),
        ],
        out_specs=pl.BlockSpec((1, OUT, BBT), lambda l, j: (l, 0, j)),
        out_shape=jax.ShapeDtypeStruct((hist, OUT, batch), jnp.float32),
    )(x_packed, W1, b1, W2, b2)


def kernel(cards_id, emb_table, W1, b1, W2, b2):
    batch, hist = cards_id.shape
    assert batch % (NW * UNIT) == 0 and batch % BBT == 0
    idx_t = cards_id.T.astype(jnp.int32)
    table_rows = emb_table.shape[0]
    ntile = table_rows // 128
    tail = table_rows - ntile * 128
    tail_p = emb_table[ntile * 128:].reshape(tail * EMB // LANE, LANE)
    fmt_k = _make_format(table_rows)
    table_p = fmt_k(emb_table.T, tail_p)
    table_c = table_p.reshape(table_rows, EMB)
    gather_k = _make_gather(batch, hist)
    packed = gather_k(table_c, idx_t)
    out_t = _mlp(
        packed, W1, b1.reshape(1, HID), W2, b2.reshape(1, OUT), batch, hist,
    )
    return out_t.transpose(2, 0, 1)


# speedup vs baseline: 1.4460x; 1.4460x over previous
import functools

import jax
import jax.numpy as jnp
from jax import lax
from jax.experimental import pallas as pl
from jax.experimental.pallas import tpu as pltpu
from jax.experimental.pallas import tpu_sc as plsc

NC = 2
NS = 16
NW = NC * NS
EMB = 32
HID = 64
OUT = 32
UNIT = 128
LANE = 128
GRP = LANE // EMB
MROW = 1024
BBT = GRP * MROW


def _make_format(table_rows: int):
    ntile = table_rows // 128
    tail = table_rows - ntile * 128
    mesh = plsc.VectorSubcoreMesh(
        core_axis_name="c", subcore_axis_name="s", num_cores=NC, num_subcores=NS
    )

    @functools.partial(
        pl.kernel,
        out_type=jax.ShapeDtypeStruct((table_rows * EMB // LANE, LANE), jnp.float32),
        mesh=mesh,
        scratch_types=[
            pltpu.VMEM((2, EMB, LANE), jnp.float32),
            pltpu.VMEM((2, EMB, LANE), jnp.float32),
            pltpu.SemaphoreType.DMA,
            pltpu.SemaphoreType.DMA,
        ],
        compiler_params=pltpu.CompilerParams(
            use_tc_tiling_on_sc=True, needs_layout_passes=False
        ),
    )
    def fmt_k(src_hbm, tail_hbm, out_hbm, tile_v, tr_v, sem0, sem1):
        wid = lax.axis_index("s") * NC + lax.axis_index("c")
        sems = (sem0, sem1)
        iota = lax.iota(jnp.int32, 16)
        nfull = ntile // NW
        nextra = ntile - nfull * NW

        def tile_of(i):
            return wid + i * NW

        def transpose_tile(b):
            @plsc.parallel_loop(0, EMB * 8, unroll=8)
            def _(k):
                q = k // 8
                p = k % 8
                f_idx = iota + (p % 2) * 16
                l_idx = jnp.broadcast_to(4 * q + p // 2, (16,)).astype(jnp.int32)
                vals = plsc.load_gather(tile_v.at[b], [f_idx, l_idx])
                tr_v[b, q, pl.ds(p * 16, 16)] = vals

        def issue(i, b):
            t = tile_of(i)
            pltpu.async_copy(
                src_hbm.at[:, pl.ds(t * LANE, LANE)], tile_v.at[b], sems[b]
            )

        def wait(i, b):
            t = tile_of(i)
            pltpu.make_async_copy(
                src_hbm.at[:, pl.ds(t * LANE, LANE)], tile_v.at[b], sems[b]
            ).wait()

        def flush(i, b):
            t = tile_of(i)
            pltpu.sync_copy(
                tr_v.at[b], out_hbm.at[pl.ds(t * EMB, EMB)]
            )

        has_extra = wid < nextra
        issue(0, 0)
        issue(1, 1)

        def body(tpair, carry):
            for b in range(2):
                i = 2 * tpair + b
                wait(i, b)
                transpose_tile(b)

                @pl.when((i + 2 < nfull) | ((i + 2 == nfull) & has_extra))
                def _():
                    issue(i + 2, b)

                flush(i, b)
            return carry

        lax.fori_loop(0, nfull // 2, body, 0)

        @pl.when(has_extra)
        def _():
            wait(nfull, 0)
            transpose_tile(0)
            flush(nfull, 0)

        if tail:
            @pl.when(wid == 0)
            def _():
                pltpu.sync_copy(
                    tail_hbm,
                    out_hbm.at[pl.ds(ntile * EMB, tail * EMB // LANE)],
                )

    return fmt_k


def _make_gather(batch: int, hist: int):
    n_rows = batch * hist
    cols_per_w = batch // NW
    tunits = cols_per_w // UNIT
    nunit = hist * tunits
    nj = batch // BBT
    mesh = plsc.VectorSubcoreMesh(
        core_axis_name="c", subcore_axis_name="s", num_cores=NC, num_subcores=NS
    )

    @functools.partial(
        pl.kernel,
        out_type=jax.ShapeDtypeStruct((n_rows // GRP, LANE), jnp.float32),
        mesh=mesh,
        scratch_types=[
            pltpu.VMEM((hist, cols_per_w), jnp.int32),
            pltpu.VMEM((2, UNIT, EMB), jnp.float32),
            pltpu.SemaphoreType.DMA,
            pltpu.SemaphoreType.DMA,
        ],
        compiler_params=pltpu.CompilerParams(use_tc_tiling_on_sc=False),
    )
    def gather_k(table_hbm, idx_hbm, out_hbm, idx_v, rows_v, sem0, sem1):
        wid = lax.axis_index("s") * NC + lax.axis_index("c")
        col0 = wid * cols_per_w
        jblk = wid // 8
        jgrp = (wid % 8) // 2
        half = (wid % 2) * 512
        lane0 = jgrp * EMB
        pltpu.sync_copy(idx_hbm.at[:, pl.ds(col0, cols_per_w)], idx_v)
        sems = (sem0, sem1)

        def unit_src(u):
            l = u // tunits
            t = u % tunits
            return table_hbm.at[idx_v.at[l, pl.ds(t * UNIT, UNIT)]]

        def unit_dst(u):
            l = u // tunits
            t = u % tunits
            q0 = (l * nj + jblk) * MROW + half + t * UNIT
            return out_hbm.at[pl.ds(q0, UNIT), pl.ds(lane0, EMB)]

        for b in range(2):
            pltpu.async_copy(unit_src(b), rows_v.at[b], sems[b])

        def body(t, carry):
            for b in range(2):
                u = 2 * t + b
                pltpu.make_async_copy(unit_src(u), rows_v.at[b], sems[b]).wait()
                pltpu.sync_copy(rows_v.at[b], unit_dst(u))
                pltpu.async_copy(unit_src(u + 2), rows_v.at[b], sems[b])
            return carry

        lax.fori_loop(0, nunit // 2 - 1, body, 0)

        for b in range(2):
            u = nunit - 2 + b
            pltpu.make_async_copy(unit_src(u), rows_v.at[b], sems[b]).wait()
            pltpu.sync_copy(rows_v.at[b], unit_dst(u))

    return gather_k


def _mlp_body(x_ref, w1_ref, b1_ref, w2_ref, b2_ref, o_ref):
    xp = x_ref[...]
    w1 = w1_ref[...]
    b1 = b1_ref[...]
    w2 = w2_ref[...]
    b2 = b2_ref[...]
    for j in range(GRP):
        x = xp[:, j * EMB:(j + 1) * EMB]
        pre1 = jnp.dot(x, w1, preferred_element_type=jnp.float32) + b1
        h = 1.0 / (1.0 + jnp.exp(-pre1))
        pre2 = jnp.dot(h, w2, preferred_element_type=jnp.float32) + b2
        y = 1.0 / (1.0 + jnp.exp(-pre2))
        o_ref[0, :, pl.ds(j * MROW, MROW)] = y.T


def _mlp(x_packed, W1, b1, W2, b2, batch, hist):
    nj = batch // BBT
    return pl.pallas_call(
        _mlp_body,
        grid=(hist, nj),
        in_specs=[
            pl.BlockSpec((MROW, LANE), lambda l, j: (l * nj + j, 0)),
            pl.BlockSpec((EMB, HID), lambda l, j: (0, 0)),
            pl.BlockSpec((1, HID), lambda l, j: (0, 0)),
            pl.BlockSpec((HID, OUT), lambda l, j: (0, 0)),
            pl.BlockSpec((1, OUT), lambda l, j: (0, 0)),
        ],
        out_specs=pl.BlockSpec((1, OUT, BBT), lambda l, j: (l, 0, j)),
        out_shape=jax.ShapeDtypeStruct((hist, OUT, batch), jnp.float32),
    )(x_packed, W1, b1, W2, b2)


def kernel(cards_id, emb_table, W1, b1, W2, b2):
    batch, hist = cards_id.shape
    assert batch % (NW * UNIT) == 0 and batch % BBT == 0
    idx_t = cards_id.T.astype(jnp.int32)
    table_rows = emb_table.shape[0]
    ntile = table_rows // 128
    tail = table_rows - ntile * 128
    tail_p = emb_table[ntile * 128:].reshape(tail * EMB // LANE, LANE)
    fmt_k = _make_format(table_rows)
    table_p = fmt_k(emb_table.T, tail_p)
    table_c = table_p.reshape(table_rows, EMB)
    gather_k = _make_gather(batch, hist)
    packed = gather_k(table_c, idx_t)
    out_t = _mlp(
        packed, W1, b1.reshape(1, HID), W2, b2.reshape(1, OUT), batch, hist,
    )
    return out_t.transpose(2, 0, 1)


# speedup vs baseline: 1.5263x; 1.0555x over previous
import functools

import jax
import jax.numpy as jnp
from jax import lax
from jax.experimental import pallas as pl
from jax.experimental.pallas import tpu as pltpu
from jax.experimental.pallas import tpu_sc as plsc

NC = 2
NS = 16
NW = NC * NS
EMB = 32
HID = 64
OUT = 32
UNIT = 128
LANE = 128
GRP = LANE // EMB
MROW = 1024
BBT = GRP * MROW


def _make_format(table_rows: int):
    ntile = table_rows // 128
    tail = table_rows - ntile * 128
    mesh = plsc.VectorSubcoreMesh(
        core_axis_name="c", subcore_axis_name="s", num_cores=NC, num_subcores=NS
    )

    @functools.partial(
        pl.kernel,
        out_type=jax.ShapeDtypeStruct((table_rows * EMB // LANE, LANE), jnp.float32),
        mesh=mesh,
        scratch_types=[
            pltpu.VMEM((2, EMB, LANE), jnp.float32),
            pltpu.VMEM((2, EMB, LANE), jnp.float32),
            pltpu.SemaphoreType.DMA,
            pltpu.SemaphoreType.DMA,
            pltpu.SemaphoreType.DMA,
            pltpu.SemaphoreType.DMA,
        ],
        compiler_params=pltpu.CompilerParams(
            use_tc_tiling_on_sc=True, needs_layout_passes=False
        ),
    )
    def fmt_k(src_hbm, tail_hbm, out_hbm, tile_v, tr_v, sem0, sem1, fsem0, fsem1):
        wid = lax.axis_index("s") * NC + lax.axis_index("c")
        sems = (sem0, sem1)
        iota = lax.iota(jnp.int32, 16)
        nfull = ntile // NW
        nextra = ntile - nfull * NW

        def tile_of(i):
            return wid + i * NW

        def transpose_tile(b):
            @plsc.parallel_loop(0, EMB * 8, unroll=8)
            def _(k):
                q = k // 8
                p = k % 8
                f_idx = iota + (p % 2) * 16
                l_idx = jnp.broadcast_to(4 * q + p // 2, (16,)).astype(jnp.int32)
                vals = plsc.load_gather(tile_v.at[b], [f_idx, l_idx])
                tr_v[b, q, pl.ds(p * 16, 16)] = vals

        def issue(i, b):
            t = tile_of(i)
            pltpu.async_copy(
                src_hbm.at[:, pl.ds(t * LANE, LANE)], tile_v.at[b], sems[b]
            )

        def wait(i, b):
            t = tile_of(i)
            pltpu.make_async_copy(
                src_hbm.at[:, pl.ds(t * LANE, LANE)], tile_v.at[b], sems[b]
            ).wait()

        fsems = (fsem0, fsem1)

        def flush(i, b):
            t = tile_of(i)
            pltpu.async_copy(
                tr_v.at[b], out_hbm.at[pl.ds(t * EMB, EMB)], fsems[b]
            )

        def wait_flush(b):
            pltpu.make_async_copy(
                tr_v.at[b], out_hbm.at[pl.ds(0, EMB)], fsems[b]
            ).wait()

        has_extra = wid < nextra
        issue(0, 0)
        issue(1, 1)

        def body(tpair, carry):
            for b in range(2):
                i = 2 * tpair + b
                wait(i, b)

                @pl.when(i >= 2)
                def _():
                    wait_flush(b)

                transpose_tile(b)

                @pl.when((i + 2 < nfull) | ((i + 2 == nfull) & has_extra))
                def _():
                    issue(i + 2, b)

                flush(i, b)
            return carry

        lax.fori_loop(0, nfull // 2, body, 0)

        @pl.when(has_extra)
        def _():
            wait(nfull, 0)
            wait_flush(0)
            transpose_tile(0)
            flush(nfull, 0)

        wait_flush(0)
        wait_flush(1)

        if tail:
            @pl.when(wid == 0)
            def _():
                pltpu.sync_copy(
                    tail_hbm,
                    out_hbm.at[pl.ds(ntile * EMB, tail * EMB // LANE)],
                )

    return fmt_k


def _make_gather(batch: int, hist: int):
    n_rows = batch * hist
    cols_per_w = batch // NW
    tunits = cols_per_w // UNIT
    nunit = hist * tunits
    nj = batch // BBT
    mesh = plsc.VectorSubcoreMesh(
        core_axis_name="c", subcore_axis_name="s", num_cores=NC, num_subcores=NS
    )

    @functools.partial(
        pl.kernel,
        out_type=jax.ShapeDtypeStruct((n_rows // GRP, LANE), jnp.float32),
        mesh=mesh,
        scratch_types=[
            pltpu.VMEM((hist, cols_per_w), jnp.int32),
            pltpu.VMEM((2, UNIT, EMB), jnp.float32),
            pltpu.SemaphoreType.DMA,
            pltpu.SemaphoreType.DMA,
        ],
        compiler_params=pltpu.CompilerParams(use_tc_tiling_on_sc=False),
    )
    def gather_k(table_hbm, idx_hbm, out_hbm, idx_v, rows_v, sem0, sem1):
        wid = lax.axis_index("s") * NC + lax.axis_index("c")
        col0 = wid * cols_per_w
        jblk = wid // 8
        jgrp = (wid % 8) // 2
        half = (wid % 2) * 512
        lane0 = jgrp * EMB
        pltpu.sync_copy(idx_hbm.at[:, pl.ds(col0, cols_per_w)], idx_v)
        sems = (sem0, sem1)

        def unit_src(u):
            l = u // tunits
            t = u % tunits
            return table_hbm.at[idx_v.at[l, pl.ds(t * UNIT, UNIT)]]

        def unit_dst(u):
            l = u // tunits
            t = u % tunits
            q0 = (l * nj + jblk) * MROW + half + t * UNIT
            return out_hbm.at[pl.ds(q0, UNIT), pl.ds(lane0, EMB)]

        for b in range(2):
            pltpu.async_copy(unit_src(b), rows_v.at[b], sems[b])

        def body(t, carry):
            for b in range(2):
                u = 2 * t + b
                pltpu.make_async_copy(unit_src(u), rows_v.at[b], sems[b]).wait()
                pltpu.sync_copy(rows_v.at[b], unit_dst(u))
                pltpu.async_copy(unit_src(u + 2), rows_v.at[b], sems[b])
            return carry

        lax.fori_loop(0, nunit // 2 - 1, body, 0)

        for b in range(2):
            u = nunit - 2 + b
            pltpu.make_async_copy(unit_src(u), rows_v.at[b], sems[b]).wait()
            pltpu.sync_copy(rows_v.at[b], unit_dst(u))

    return gather_k


def _mlp_body(x_ref, w1_ref, b1_ref, w2_ref, b2_ref, o_ref):
    xp = x_ref[...]
    w1 = w1_ref[...]
    b1 = b1_ref[...]
    w2 = w2_ref[...]
    b2 = b2_ref[...]
    for j in range(GRP):
        x = xp[:, j * EMB:(j + 1) * EMB]
        pre1 = jnp.dot(x, w1, preferred_element_type=jnp.float32) + b1
        h = 1.0 / (1.0 + jnp.exp(-pre1))
        pre2 = jnp.dot(h, w2, preferred_element_type=jnp.float32) + b2
        y = 1.0 / (1.0 + jnp.exp(-pre2))
        o_ref[0, :, pl.ds(j * MROW, MROW)] = y.T


def _mlp(x_packed, W1, b1, W2, b2, batch, hist):
    nj = batch // BBT
    return pl.pallas_call(
        _mlp_body,
        grid=(hist, nj),
        in_specs=[
            pl.BlockSpec((MROW, LANE), lambda l, j: (l * nj + j, 0)),
            pl.BlockSpec((EMB, HID), lambda l, j: (0, 0)),
            pl.BlockSpec((1, HID), lambda l, j: (0, 0)),
            pl.BlockSpec((HID, OUT), lambda l, j: (0, 0)),
            pl.BlockSpec((1, OUT), lambda l, j: (0, 0)),
        ],
        out_specs=pl.BlockSpec((1, OUT, BBT), lambda l, j: (l, 0, j)),
        out_shape=jax.ShapeDtypeStruct((hist, OUT, batch), jnp.float32),
    )(x_packed, W1, b1, W2, b2)


def kernel(cards_id, emb_table, W1, b1, W2, b2):
    batch, hist = cards_id.shape
    assert batch % (NW * UNIT) == 0 and batch % BBT == 0
    idx_t = cards_id.T.astype(jnp.int32)
    table_rows = emb_table.shape[0]
    ntile = table_rows // 128
    tail = table_rows - ntile * 128
    tail_p = emb_table[ntile * 128:].reshape(tail * EMB // LANE, LANE)
    fmt_k = _make_format(table_rows)
    table_p = fmt_k(emb_table.T, tail_p)
    table_c = table_p.reshape(table_rows, EMB)
    gather_k = _make_gather(batch, hist)
    packed = gather_k(table_c, idx_t)
    out_t = _mlp(
        packed, W1, b1.reshape(1, HID), W2, b2.reshape(1, OUT), batch, hist,
    )
    return out_t.transpose(2, 0, 1)
